# baseline (speedup 1.0000x reference)
import functools

import jax
import jax.numpy as jnp
from jax import lax
from jax.experimental import pallas as pl
from jax.experimental.pallas import tpu as pltpu
from jax.experimental.pallas import tpu_sc as plsc

B, L, D = 256, 144, 768
N_PATCH = 576
NROWS_TBL = N_PATCH + 1
R = B * L
NC, NS, LANES = 2, 16, 16
DQ = 128
NDQ = D // DQ
NRG = 4
NACT = NRG * NDQ
ROWS_PER_RG = R // NRG
C = 64
N_CHUNKS = ROWS_PER_RG // C
NBUF = 3
VPR = DQ // LANES

_mesh = plsc.VectorSubcoreMesh(core_axis_name="c", subcore_axis_name="s")


@functools.partial(
    pl.kernel,
    out_type=jax.ShapeDtypeStruct((R, D), jnp.float32),
    mesh=_mesh,
    scratch_types=dict(
        tbuf=pltpu.VMEM((NROWS_TBL, DQ), jnp.float32),
        idx_all=pltpu.VMEM((ROWS_PER_RG,), jnp.int32),
        bufs=[pltpu.VMEM((C, DQ), jnp.float32) for _ in range(NBUF)],
        xsems=pltpu.SemaphoreType.DMA((NBUF,)),
        ssems=pltpu.SemaphoreType.DMA((NBUF,)),
    ),
)
def _pe_add_kernel(x_hbm, idx_hbm, table_hbm, out_hbm, *, tbuf, idx_all,
                   bufs, xsems, ssems):
    wid = lax.axis_index("s") * NC + lax.axis_index("c")

    @pl.when(wid < NACT)
    def _work():
        rg = wid // NDQ
        dq = wid - rg * NDQ
        col0 = pl.multiple_of(dq * DQ, DQ)
        row0 = pl.multiple_of(rg * ROWS_PER_RG, 8)

        pltpu.sync_copy(table_hbm.at[:, pl.ds(col0, DQ)], tbuf)
        pltpu.sync_copy(idx_hbm.at[pl.ds(row0, ROWS_PER_RG)], idx_all)

        def xcopy_desc(ci, k):
            return pltpu.make_async_copy(
                x_hbm.at[pl.ds(row0 + ci * C, C), pl.ds(col0, DQ)],
                bufs[k], xsems.at[k])

        def store_desc(ci, k):
            return pltpu.make_async_copy(
                bufs[k], out_hbm.at[pl.ds(row0 + ci * C, C),
                                    pl.ds(col0, DQ)], ssems.at[k])

        def prefetch(ci, k, first_round):
            if not first_round:
                store_desc(ci, k).wait()
            xcopy_desc(ci, k).start()

        def process(ci, k):
            xcopy_desc(ci, k).wait()
            for g in range(C // LANES):
                iv = idx_all[pl.ds(ci * C + g * LANES, LANES)]
                for r in range(LANES):
                    t = iv[r] + 1
                    for v in range(VPR):
                        sl = pl.ds(v * LANES, LANES)
                        plsc.addupdate(bufs[k].at[g * LANES + r, sl],
                                       tbuf[t, sl])
            store_desc(ci, k).start()

        prefetch(0, 0, True)

        @pl.loop(0, N_CHUNKS, step=NBUF)
        def _main(ci0):
            for k in range(NBUF):
                ci = ci0 + k
                kn = (k + 1) % NBUF

                @pl.when(jnp.logical_and(ci + 1 < N_CHUNKS,
                                         ci >= NBUF - 1))
                def _pf():
                    store_desc(ci + 1, kn).wait()
                    xcopy_desc(ci + 1, kn).start()

                @pl.when(ci + 1 < NBUF - 1 + 1)
                def _pf_nowait():
                    xcopy_desc(ci + 1, kn).start()

                process(ci, k)

        store_desc(N_CHUNKS - 3, 0).wait()
        store_desc(N_CHUNKS - 2, 1).wait()
        store_desc(N_CHUNKS - 1, 2).wait()


def kernel(unmask_patch_embed, unmask_idx, cls_encode, pe_encode):
    del cls_encode
    x = unmask_patch_embed.reshape(R, D)
    idx = unmask_idx.reshape(R).astype(jnp.int32)
    table = pe_encode.reshape(NROWS_TBL, D)
    out = _pe_add_kernel(x, idx, table)
    return out.reshape(B, L, D)


# speedup vs baseline: 2.7181x; 2.7181x over previous
import functools

import jax
import jax.numpy as jnp
from jax import lax
from jax.experimental import pallas as pl
from jax.experimental.pallas import tpu as pltpu
from jax.experimental.pallas import tpu_sc as plsc

B, L, D = 256, 144, 768
N_PATCH = 576
NROWS_TBL = N_PATCH + 1
R = B * L
NC, NS, LANES = 2, 16, 16
NW = NC * NS
ROWS_PER_W = R // NW
C = 24
N_CHUNKS = ROWS_PER_W // C
NBUF = 3
VPR = D // LANES

_mesh = plsc.VectorSubcoreMesh(core_axis_name="c", subcore_axis_name="s")


@functools.partial(
    pl.kernel,
    out_type=jax.ShapeDtypeStruct((R, D), jnp.float32),
    mesh=_mesh,
    scratch_types=dict(
        idx_all=pltpu.VMEM((ROWS_PER_W,), jnp.int32),
        xs=[pltpu.VMEM((C, D), jnp.float32) for _ in range(NBUF)],
        rows=[pltpu.VMEM((C, D), jnp.float32) for _ in range(NBUF)],
        gsems=pltpu.SemaphoreType.DMA((NBUF,)),
        xsems=pltpu.SemaphoreType.DMA((NBUF,)),
        ssems=pltpu.SemaphoreType.DMA((NBUF,)),
    ),
)
def _pe_add_kernel(x_hbm, idx_hbm, table_hbm, out_hbm, *, idx_all,
                   xs, rows, gsems, xsems, ssems):
    sid = lax.axis_index("s")
    wid = sid * NC + lax.axis_index("c")
    base0 = wid * ROWS_PER_W

    pltpu.sync_copy(idx_hbm.at[pl.ds(base0, ROWS_PER_W)], idx_all)

    @pl.loop(0, ROWS_PER_W // LANES, unroll=8)
    def _inc(j):
        sl = pl.ds(j * LANES, LANES)
        idx_all[sl] = idx_all[sl] + 1

    def gather_desc(ci, k):
        return pltpu.make_async_copy(
            table_hbm.at[idx_all.at[pl.ds(ci * C, C)]], rows[k], gsems.at[k])

    def xcopy_desc(ci, k):
        return pltpu.make_async_copy(
            x_hbm.at[pl.ds(base0 + ci * C, C)], xs[k], xsems.at[k])

    def store_desc(ci, k):
        return pltpu.make_async_copy(
            rows[k], out_hbm.at[pl.ds(base0 + ci * C, C)], ssems.at[k])

    def prefetch(ci, k, wait_store):
        if wait_store:
            store_desc(ci, k).wait()
        gather_desc(ci, k).start()
        xcopy_desc(ci, k).start()

    def process(ci, k):
        gather_desc(ci, k).wait()
        xcopy_desc(ci, k).wait()

        @pl.loop(0, C)
        def _row(r):
            for v in range(VPR):
                sl = pl.ds(v * LANES, LANES)
                plsc.addupdate(rows[k].at[r, sl], xs[k][r, sl])

        store_desc(ci, k).start()

    prefetch(0, 0, False)
    prefetch(1, 1, False)
    process(0, 0)
    prefetch(2, 2, False)
    process(1, 1)
    prefetch(3, 0, True)
    process(2, 2)

    @pl.loop(NBUF, N_CHUNKS - NBUF, step=NBUF)
    def _main(ci):
        for k in range(NBUF):
            prefetch(ci + k + 1, (k + 1) % NBUF, True)
            process(ci + k, k)

    prefetch(N_CHUNKS - 2, 1, True)
    process(N_CHUNKS - 3, 0)
    prefetch(N_CHUNKS - 1, 2, True)
    process(N_CHUNKS - 2, 1)
    process(N_CHUNKS - 1, 2)
    store_desc(N_CHUNKS - 3, 0).wait()
    store_desc(N_CHUNKS - 2, 1).wait()
    store_desc(N_CHUNKS - 1, 2).wait()


def kernel(unmask_patch_embed, unmask_idx, cls_encode, pe_encode):
    del cls_encode
    x = unmask_patch_embed.reshape(R, D)
    idx = unmask_idx.reshape(R).astype(jnp.int32)
    table = pe_encode.reshape(NROWS_TBL, D)
    out = _pe_add_kernel(x, idx, table)
    return out.reshape(B, L, D)
